# baseline (speedup 1.0000x reference)
import functools

import jax
import jax.numpy as jnp
from jax import lax
from jax.experimental import pallas as pl
from jax.experimental.pallas import tpu as pltpu
from jax.experimental.pallas import tpu_sc as plsc


def kernel(idx, emb_weight):
    B = idx.shape[0]
    V, D = emb_weight.shape
    T2 = emb_weight.reshape(V // 2, 2 * D)

    info = plsc.get_sparse_core_info()
    NC, NS = info.num_cores, info.num_subcores
    NW = NC * NS
    assert B % NW == 0
    bpw = B // NW

    pidx = idx // 2
    mesh = plsc.VectorSubcoreMesh(core_axis_name="c", subcore_axis_name="s")

    @functools.partial(
        pl.kernel,
        mesh=mesh,
        out_type=jax.ShapeDtypeStruct((B, 2 * D), jnp.float32),
        scratch_types=[
            pltpu.VMEM((bpw,), jnp.int32),
            pltpu.VMEM((bpw, 2 * D), jnp.float32),
            pltpu.SemaphoreType.DMA,
        ],
        compiler_params=pltpu.CompilerParams(use_tc_tiling_on_sc=True),
    )
    def gather_pairs(idx_hbm, t_hbm, out_hbm, idx_v, pairs_v, sem):
        wid = lax.axis_index("s") * NC + lax.axis_index("c")
        base = wid * bpw
        pltpu.sync_copy(idx_hbm.at[pl.ds(base, bpw)], idx_v)
        pltpu.async_copy(t_hbm.at[idx_v], pairs_v, sem).wait()
        pltpu.sync_copy(pairs_v, out_hbm.at[pl.ds(base, bpw)])

    wide = gather_pairs(pidx.astype(jnp.int32), T2)
    return jnp.where((idx % 2 == 1)[:, None], wide[:, D:], wide[:, :D])
